# baseline (speedup 1.0000x reference)
import functools

import jax
import jax.numpy as jnp
from jax import lax
from jax.experimental import pallas as pl
from jax.experimental.pallas import tpu as pltpu
from jax.experimental.pallas import tpu_sc as plsc

GRAPH_NUM = 1000000
EMB_DIM = 64
OUT_DIM = 64
BATCH = 16384

NUM_CORES = 2
NUM_SUBCORES = 16
NW = NUM_CORES * NUM_SUBCORES
GRP = 16

CW = 512
PHYS_MINOR = 1000064
N_CH = -(-PHYS_MINOR // CW)
LAST_OFF = PHYS_MINOR - CW
M_PER_W = -(-N_CH // NW)
N_BODY = -(-M_PER_W // 2)
ST_PAD = N_CH + 2 * GRP
DUMP = BATCH
OUT_ROWS = BATCH + GRP


def _make_sc_gather():
    mesh = plsc.VectorSubcoreMesh(core_axis_name="c", subcore_axis_name="s")

    @functools.partial(
        pl.kernel,
        mesh=mesh,
        compiler_params=pltpu.CompilerParams(
            use_tc_tiling_on_sc=True, needs_layout_passes=False),
        out_type=jax.ShapeDtypeStruct((OUT_ROWS, 128), jnp.float32),
        scratch_types=[
            pltpu.VMEM((BATCH + GRP,), jnp.int32),
            pltpu.VMEM((BATCH + GRP,), jnp.int32),
            pltpu.VMEM((ST_PAD,), jnp.int32),
            pltpu.VMEM((GRP, 128), jnp.float32),
            pltpu.VMEM((EMB_DIM, CW), jnp.float32),
            pltpu.VMEM((EMB_DIM, CW), jnp.float32),
            pltpu.SemaphoreType.DMA,
            pltpu.SemaphoreType.DMA,
            pltpu.SemaphoreType.DMA,
        ],
    )
    def gather(tT_hbm, gsort_hbm, pos_hbm, starts_hbm, out_hbm,
               gs_v, ps_v, st_v, stage, b0, b1, s0, s1, ssem):
        wid = lax.axis_index("s") * NUM_CORES + lax.axis_index("c")
        pltpu.sync_copy(gsort_hbm, gs_v)
        pltpu.sync_copy(pos_hbm, ps_v)
        pltpu.sync_copy(starts_hbm, st_v)
        lanes = lax.iota(jnp.int32, GRP)

        def chunk_of(m):
            return jnp.minimum(m * NW + wid, N_CH - 1)

        def fire(m, buf, sem):
            off = pl.multiple_of(
                jnp.minimum(chunk_of(m) * CW, LAST_OFF), 128)
            return pltpu.async_copy(
                tT_hbm.at[:, pl.ds(off, CW)], buf, sem)

        def process(m, buf):
            c = chunk_of(m)
            off = jnp.minimum(c * CW, LAST_OFF)
            svec = st_v[pl.ds(c, GRP)]
            s = lax.reduce_sum(jnp.where(lanes == 0, svec, 0), axes=(0,))
            e = lax.reduce_sum(jnp.where(lanes == 1, svec, 0), axes=(0,))
            n_g = (e - s + GRP - 1) >> 4
            off_v = jnp.full((GRP,), off, jnp.int32)

            def grp_body(j, carry):
                h = s + j * GRP
                gv = gs_v[pl.ds(h, GRP)]
                pv = ps_v[pl.ds(h, GRP)]
                valid = (h + lanes) < e
                local = jnp.where(valid, gv - off_v, 0)
                pos = jnp.where(valid, pv, DUMP + lanes)
                for eg in range(EMB_DIM):
                    row = jnp.full((GRP,), eg, jnp.int32)
                    vals = plsc.load_gather(buf, [row, local], mask=valid)
                    plsc.store_scatter(stage, [lanes, row], vals, mask=valid)
                pltpu.async_copy(stage, out_hbm.at[pos], ssem).wait()
                return carry

            lax.fori_loop(0, n_g, grp_body, 0)

        fire(0, b0, s0)
        fire(1, b1, s1)

        @pl.loop(0, N_BODY)
        def _body(r):
            m0 = 2 * r
            pltpu.make_async_copy(
                tT_hbm.at[:, pl.ds(pl.multiple_of(0, 128), CW)], b0, s0
            ).wait()
            process(m0, b0)
            fire(m0 + 2, b0, s0)
            pltpu.make_async_copy(
                tT_hbm.at[:, pl.ds(pl.multiple_of(0, 128), CW)], b1, s1
            ).wait()
            process(m0 + 1, b1)
            fire(m0 + 3, b1, s1)

        pltpu.make_async_copy(
            tT_hbm.at[:, pl.ds(pl.multiple_of(0, 128), CW)], b0, s0).wait()
        pltpu.make_async_copy(
            tT_hbm.at[:, pl.ds(pl.multiple_of(0, 128), CW)], b1, s1).wait()

    return gather


_sc_gather = _make_sc_gather()


def _proj_body(emb_ref, w_ref, outT_ref):
    e = emb_ref[...][:, :EMB_DIM]
    yT = lax.dot_general(
        w_ref[...], e,
        dimension_numbers=(((1,), (1,)), ((), ())),
        preferred_element_type=jnp.float32,
    )
    norm = jnp.sqrt(jnp.sum(yT * yT, axis=0, keepdims=True))
    outT_ref[...] = yT / jnp.maximum(norm, 1e-12)


_PROJ_BLOCK = 2048


def _tc_project(emb3, W):
    grid = (BATCH // _PROJ_BLOCK,)
    return pl.pallas_call(
        _proj_body,
        grid=grid,
        in_specs=[
            pl.BlockSpec((_PROJ_BLOCK, 128), lambda i: (i, 0)),
            pl.BlockSpec((OUT_DIM, EMB_DIM), lambda i: (0, 0)),
        ],
        out_specs=pl.BlockSpec((OUT_DIM, _PROJ_BLOCK), lambda i: (0, i)),
        out_shape=jax.ShapeDtypeStruct((OUT_DIM, BATCH), jnp.float32),
    )(emb3, W)


def kernel(graph_id_list, table, W):
    idx = graph_id_list.astype(jnp.int32)
    tableT = table.T
    c_id = idx // CW
    order = jnp.argsort(c_id, stable=True).astype(jnp.int32)
    gsort = jnp.concatenate([idx[order], jnp.zeros((GRP,), jnp.int32)])
    pos = jnp.concatenate([order, jnp.full((GRP,), DUMP, jnp.int32)])
    starts = jnp.searchsorted(
        c_id[order], jnp.arange(N_CH + 1, dtype=jnp.int32)).astype(jnp.int32)
    starts = jnp.concatenate(
        [starts, jnp.full((ST_PAD - N_CH - 1,), BATCH, jnp.int32)])
    emb3 = _sc_gather(tableT, gsort, pos, starts)
    outT = _tc_project(emb3, W)
    return outT.T


# speedup vs baseline: 1.4707x; 1.4707x over previous
import functools

import jax
import jax.numpy as jnp
from jax import lax
from jax.experimental import pallas as pl
from jax.experimental.pallas import tpu as pltpu
from jax.experimental.pallas import tpu_sc as plsc

GRAPH_NUM = 1000000
EMB_DIM = 64
OUT_DIM = 64
BATCH = 16384

NUM_CORES = 2
NUM_SUBCORES = 16
NW = NUM_CORES * NUM_SUBCORES
GRP = 16

CW = 384
PHYS_MINOR = 1000064
N_CH = -(-PHYS_MINOR // CW)
LAST_OFF = PHYS_MINOR - CW
M_PER_W = -(-N_CH // NW)
NRING = 3
N_BODY = -(-(M_PER_W + 1) // NRING)
DUMP = BATCH
OUT_ROWS = BATCH + GRP
N_IVEC = BATCH // GRP


def _make_sc_gather():
    mesh = plsc.VectorSubcoreMesh(core_axis_name="c", subcore_axis_name="s")

    @functools.partial(
        pl.kernel,
        mesh=mesh,
        compiler_params=pltpu.CompilerParams(
            use_tc_tiling_on_sc=True, needs_layout_passes=False),
        out_type=jax.ShapeDtypeStruct((OUT_ROWS, 128), jnp.float32),
        scratch_types=[
            pltpu.VMEM((BATCH,), jnp.int32),
            pltpu.VMEM((BATCH + GRP,), jnp.int32),
            pltpu.VMEM((BATCH + GRP,), jnp.int32),
            pltpu.VMEM((2 * GRP,), jnp.int32),
            pltpu.VMEM((2 * GRP,), jnp.int32),
            pltpu.VMEM((GRP, 128), jnp.float32),
            pltpu.VMEM((EMB_DIM, CW), jnp.float32),
            pltpu.VMEM((EMB_DIM, CW), jnp.float32),
            pltpu.VMEM((EMB_DIM, CW), jnp.float32),
            pltpu.SemaphoreType.DMA,
            pltpu.SemaphoreType.DMA,
            pltpu.SemaphoreType.DMA,
            pltpu.SemaphoreType.DMA,
        ],
    )
    def gather(tT_hbm, idx_hbm, out_hbm,
               aidx, hg, hp, pg, pp, stage, b0, b1, b2, s0, s1, s2, ssem):
        wid = lax.axis_index("s") * NUM_CORES + lax.axis_index("c")
        bufs = (b0, b1, b2)
        sems = (s0, s1, s2)
        lanes = lax.iota(jnp.int32, GRP)
        lo = wid * (M_PER_W * CW)

        def chunk_off(m):
            c = jnp.minimum(wid * M_PER_W + m, N_CH - 1)
            return pl.multiple_of(jnp.minimum(c * CW, LAST_OFF), 128)

        def fire(m, q):
            return pltpu.async_copy(
                tT_hbm.at[:, pl.ds(chunk_off(m), CW)], bufs[q], sems[q])

        def drain(q):
            pltpu.make_async_copy(
                tT_hbm.at[:, pl.ds(pl.multiple_of(0, 128), CW)],
                bufs[q], sems[q]).wait()

        for q in range(NRING):
            fire(q, q)

        pltpu.sync_copy(idx_hbm, aidx)
        lo_v = jnp.full((GRP,), lo, jnp.int32)
        hi_v = jnp.full((GRP,), lo + M_PER_W * CW, jnp.int32)

        def scan_body(j, cur):
            vec = aidx[pl.ds(j * GRP, GRP)]
            rows = j * GRP + lanes
            m = (vec >= lo_v) & (vec < hi_v)
            cnt = lax.reduce_max(
                plsc.all_reduce_population_count(m), axes=(0,))
            plsc.store_compressed(hg.at[pl.ds(cur, GRP)], vec, mask=m)
            plsc.store_compressed(hp.at[pl.ds(cur, GRP)], rows, mask=m)
            return cur + cnt

        n_hits = lax.fori_loop(0, N_IVEC, scan_body, jnp.int32(0))
        n_hvec = (n_hits + GRP - 1) >> 4

        def emit_group(mask, off_v):
            gvec = pg[pl.ds(0, GRP)]
            pvec = pp[pl.ds(0, GRP)]
            local = jnp.where(mask, gvec - off_v, 0)
            pos = jnp.where(mask, pvec, DUMP + lanes)
            return local, pos

        def process(m, buf):
            c = jnp.minimum(wid * M_PER_W + m, N_CH - 1)
            off = jnp.minimum(c * CW, LAST_OFF)
            off_v = jnp.full((GRP,), off, jnp.int32)
            cw_v = jnp.full((GRP,), CW, jnp.int32)

            def do_group(mask, local, pos):
                for eg in range(EMB_DIM):
                    row = jnp.full((GRP,), eg, jnp.int32)
                    vals = plsc.load_gather(buf, [row, local], mask=mask)
                    plsc.store_scatter(stage, [lanes, row], vals, mask=mask)
                pltpu.async_copy(stage, out_hbm.at[pos], ssem).wait()

            def hscan(k, pcur):
                hv = hg[pl.ds(k * GRP, GRP)]
                pv = hp[pl.ds(k * GRP, GRP)]
                inb = (k * GRP + lanes) < n_hits
                rel = hv - off_v
                m2 = inb & (rel >= 0) & (rel < cw_v)
                cnt = lax.reduce_max(
                    plsc.all_reduce_population_count(m2), axes=(0,))
                plsc.store_compressed(pg.at[pl.ds(pcur, GRP)], hv, mask=m2)
                plsc.store_compressed(pp.at[pl.ds(pcur, GRP)], pv, mask=m2)
                pcur = pcur + cnt

                @pl.when(pcur >= GRP)
                def _flush_full():
                    local, pos = emit_group(lanes >= 0, off_v)
                    do_group(lanes >= 0, local, pos)
                    pg[pl.ds(0, GRP)] = pg[pl.ds(GRP, GRP)]
                    pp[pl.ds(0, GRP)] = pp[pl.ds(GRP, GRP)]

                return jnp.where(pcur >= GRP, pcur - GRP, pcur)

            pcur = lax.fori_loop(0, n_hvec, hscan, jnp.int32(0))

            @pl.when(pcur > 0)
            def _flush_tail():
                mask = lanes < pcur
                local, pos = emit_group(mask, off_v)
                do_group(mask, local, pos)

        @pl.loop(0, N_BODY)
        def _body(r):
            for q in range(NRING):
                drain(q)
                process(NRING * r + q, bufs[q])
                fire(NRING * r + q + NRING, q)

        for q in range(NRING):
            drain(q)

    return gather


_sc_gather = _make_sc_gather()


def _proj_body(emb_ref, w_ref, outT_ref):
    e = emb_ref[...][:, :EMB_DIM]
    yT = lax.dot_general(
        w_ref[...], e,
        dimension_numbers=(((1,), (1,)), ((), ())),
        preferred_element_type=jnp.float32,
    )
    norm = jnp.sqrt(jnp.sum(yT * yT, axis=0, keepdims=True))
    outT_ref[...] = yT / jnp.maximum(norm, 1e-12)


_PROJ_BLOCK = 2048


def _tc_project(emb3, W):
    grid = (BATCH // _PROJ_BLOCK,)
    return pl.pallas_call(
        _proj_body,
        grid=grid,
        in_specs=[
            pl.BlockSpec((_PROJ_BLOCK, 128), lambda i: (i, 0)),
            pl.BlockSpec((OUT_DIM, EMB_DIM), lambda i: (0, 0)),
        ],
        out_specs=pl.BlockSpec((OUT_DIM, _PROJ_BLOCK), lambda i: (0, i)),
        out_shape=jax.ShapeDtypeStruct((OUT_DIM, BATCH), jnp.float32),
    )(emb3, W)


def kernel(graph_id_list, table, W):
    idx = graph_id_list.astype(jnp.int32)
    tableT = table.T
    emb3 = _sc_gather(tableT, idx)
    outT = _tc_project(emb3, W)
    return outT.T


# speedup vs baseline: 2.1264x; 1.4459x over previous
import functools

import jax
import jax.numpy as jnp
from jax import lax
from jax.experimental import pallas as pl
from jax.experimental.pallas import tpu as pltpu
from jax.experimental.pallas import tpu_sc as plsc

GRAPH_NUM = 1000000
EMB_DIM = 64
OUT_DIM = 64
BATCH = 16384

NUM_CORES = 2
NUM_SUBCORES = 16
NW = NUM_CORES * NUM_SUBCORES
B_PER_W = BATCH // NW
GRP = 16
N_GRP = B_PER_W // GRP
NBUF = 7


def _make_sc_gather():
    mesh = plsc.VectorSubcoreMesh(core_axis_name="c", subcore_axis_name="s")

    @functools.partial(
        pl.kernel,
        mesh=mesh,
        compiler_params=pltpu.CompilerParams(
            use_tc_tiling_on_sc=True, needs_layout_passes=False),
        out_type=jax.ShapeDtypeStruct((BATCH, EMB_DIM), jnp.float32),
        scratch_types=[
            pltpu.VMEM((B_PER_W,), jnp.int32),
            pltpu.VMEM((B_PER_W, EMB_DIM), jnp.float32),
        ]
        + [pltpu.VMEM((EMB_DIM, 128), jnp.float32) for _ in range(NBUF)]
        + [pltpu.SemaphoreType.DMA for _ in range(NBUF)],
    )
    def gather(tT_hbm, idx_hbm, out_hbm, idx_v, out_v, *bufsems):
        bufs = bufsems[:NBUF]
        sems = bufsems[NBUF:]
        wid = lax.axis_index("s") * NUM_CORES + lax.axis_index("c")
        base = wid * B_PER_W
        pltpu.sync_copy(idx_hbm.at[pl.ds(base, B_PER_W)], idx_v)
        lanes = lax.iota(jnp.int32, GRP)

        @pl.loop(0, N_GRP)
        def _grp(grp):
            vec = idx_v[pl.ds(grp * GRP, GRP)]
            gs = [
                lax.reduce_sum(
                    jnp.where(lanes == j, vec, 0), axes=(0,))
                for j in range(GRP)
            ]

            def fire(i):
                off = pl.multiple_of((gs[i] >> 7) << 7, 128)
                return pltpu.async_copy(
                    tT_hbm.at[:, pl.ds(off, 128)], bufs[i % NBUF],
                    sems[i % NBUF])

            copies = [None] * GRP
            for i in range(NBUF):
                copies[i] = fire(i)
            for i in range(GRP):
                copies[i].wait()
                buf = bufs[i % NBUF]
                col = jnp.full((GRP,), gs[i] & 127, jnp.int32)
                row = grp * GRP + i
                for e in range(EMB_DIM // GRP):
                    vals = plsc.load_gather(buf, [e * GRP + lanes, col])
                    out_v[row, pl.ds(e * GRP, GRP)] = vals
                if i + NBUF < GRP:
                    copies[i + NBUF] = fire(i + NBUF)

        pltpu.sync_copy(out_v, out_hbm.at[pl.ds(base, B_PER_W)])

    return gather


_sc_gather = _make_sc_gather()


def _proj_body(emb_ref, w_ref, outT_ref):
    e = emb_ref[...]
    yT = lax.dot_general(
        w_ref[...], e,
        dimension_numbers=(((1,), (1,)), ((), ())),
        preferred_element_type=jnp.float32,
    )
    norm = jnp.sqrt(jnp.sum(yT * yT, axis=0, keepdims=True))
    outT_ref[...] = yT / jnp.maximum(norm, 1e-12)


_PROJ_BLOCK = 2048


def _tc_project(emb, W):
    grid = (BATCH // _PROJ_BLOCK,)
    return pl.pallas_call(
        _proj_body,
        grid=grid,
        in_specs=[
            pl.BlockSpec((_PROJ_BLOCK, EMB_DIM), lambda i: (i, 0)),
            pl.BlockSpec((OUT_DIM, EMB_DIM), lambda i: (0, 0)),
        ],
        out_specs=pl.BlockSpec((OUT_DIM, _PROJ_BLOCK), lambda i: (0, i)),
        out_shape=jax.ShapeDtypeStruct((OUT_DIM, BATCH), jnp.float32),
    )(emb, W)


def kernel(graph_id_list, table, W):
    idx = graph_id_list.astype(jnp.int32)
    tableT = table.T
    emb = _sc_gather(tableT, idx)
    outT = _tc_project(emb, W)
    return outT.T
